# baseline (speedup 1.0000x reference)
import functools

import jax
import jax.numpy as jnp
from jax import lax
from jax.experimental import pallas as pl
from jax.experimental.pallas import tpu as pltpu
from jax.experimental.pallas import tpu_sc as plsc

N_NODES = 10000
N_EDGES = 320000
LATENT = 128
MP_STEPS = 10
N_TYPES = 9

NC, NS = 2, 16
NW = NC * NS

F32 = jnp.float32


BF16 = jnp.bfloat16


def _split(x):
    hi = x.astype(BF16)
    lo = (x - hi.astype(F32)).astype(BF16)
    return hi, lo


def _wsplit(w):
    hi, lo = _split(w)
    return {'h': hi, 'l': lo}


def _d3(x, wh, wl):
    xh, xl = _split(x)
    d = lambda a, b: jnp.dot(a, b, preferred_element_type=F32)
    return d(xh, wh) + (d(xh, wl) + d(xl, wh))


def _mlp_tail(x, w1h, w1l, b1, w2h, w2l, b2, ln_s, ln_o):
    x = jnp.maximum(x, 0.0)
    x = _d3(x, w1h, w1l) + b1
    x = jnp.maximum(x, 0.0)
    x = _d3(x, w2h, w2l) + b2
    mu = jnp.mean(x, axis=-1, keepdims=True)
    d = x - mu
    var = jnp.mean(d * d, axis=-1, keepdims=True)
    return d * lax.rsqrt(var + 1e-5) * ln_s + ln_o


def _row_spec(block, cols):
    return pl.BlockSpec((block, cols), lambda i: (i, 0))


def _full_spec(shape):
    nd = len(shape)
    return pl.BlockSpec(shape, lambda i: (0,) * nd)


_BN = 2000
_BE = 2000


def _enc_node_call(nodes, pt2d, w0n, te, b0, w1, b1, w2, b2, ln_s, ln_o):
    def body(n_ref, pt_ref, w0h_ref, w0l_ref, te_ref, b0_ref, w1h_ref, w1l_ref,
             b1_ref, w2h_ref, w2l_ref, b2_ref, s_ref, o_ref, h_ref):
        pt = pt_ref[...]
        te = te_ref[...]
        emb = jnp.zeros((pt.shape[0], te.shape[1]), F32)
        for t in range(N_TYPES):
            emb = jnp.where(pt == t, te[t][None, :], emb)
        xc = jnp.concatenate([n_ref[...], emb], axis=-1)
        x = _d3(xc, w0h_ref[...], w0l_ref[...]) + b0_ref[...]
        h_ref[...] = _mlp_tail(x, w1h_ref[...], w1l_ref[...], b1_ref[...],
                               w2h_ref[...], w2l_ref[...], b2_ref[...],
                               s_ref[...], o_ref[...])

    args = (nodes, pt2d, w0n['h'], w0n['l'], te, b0, w1['h'], w1['l'], b1,
            w2['h'], w2['l'], b2, ln_s, ln_o)
    grid = (N_NODES // _BN,)
    return pl.pallas_call(
        body,
        grid=grid,
        in_specs=[_row_spec(_BN, 128), _row_spec(_BN, 1)] + [
            _full_spec(a.shape) for a in args[2:]],
        out_specs=_row_spec(_BN, LATENT),
        out_shape=jax.ShapeDtypeStruct((N_NODES, LATENT), F32),
    )(*args)


def _enc_edge_call(edges, w0, b0, w1, b1, w2, b2, ln_s, ln_o):
    def body(e_ref, w0h_ref, w0l_ref, b0_ref, w1h_ref, w1l_ref, b1_ref,
             w2h_ref, w2l_ref, b2_ref, s_ref, o_ref, out_ref):
        x = _d3(e_ref[...], w0h_ref[...], w0l_ref[...]) + b0_ref[...]
        out_ref[...] = _mlp_tail(x, w1h_ref[...], w1l_ref[...], b1_ref[...],
                                 w2h_ref[...], w2l_ref[...], b2_ref[...],
                                 s_ref[...], o_ref[...])

    args = (edges, w0['h'], w0['l'], b0, w1['h'], w1['l'], b1, w2['h'],
            w2['l'], b2, ln_s, ln_o)
    grid = (N_EDGES // _BE,)
    return pl.pallas_call(
        body,
        grid=grid,
        in_specs=[_row_spec(_BE, 16)] + [_full_spec(a.shape) for a in args[1:]],
        out_specs=_row_spec(_BE, LATENT),
        out_shape=jax.ShapeDtypeStruct((N_EDGES, LATENT), F32),
    )(*args)


def _proj_call(h, w0s, w0r, b0):
    def body(h_ref, wsh_ref, wsl_ref, wrh_ref, wrl_ref, b0_ref, ps_ref, pr_ref):
        hh = h_ref[...]
        ps_ref[...] = _d3(hh, wsh_ref[...], wsl_ref[...]) + b0_ref[...]
        pr_ref[...] = _d3(hh, wrh_ref[...], wrl_ref[...])

    args = (h, w0s['h'], w0s['l'], w0r['h'], w0r['l'], b0)
    grid = (N_NODES // _BN,)
    return pl.pallas_call(
        body,
        grid=grid,
        in_specs=[_row_spec(_BN, LATENT)] + [
            _full_spec(a.shape) for a in args[1:]],
        out_specs=[_row_spec(_BN, LATENT), _row_spec(_BN, LATENT)],
        out_shape=[jax.ShapeDtypeStruct((N_NODES, LATENT), F32)] * 2,
    )(*args)


def _edge_call(gs, gr, e, w0e, b0, w1, b1, w2, b2, ln_s, ln_o):
    def body(gs_ref, gr_ref, e_ref, w0h_ref, w0l_ref, b0_ref, w1h_ref,
             w1l_ref, b1_ref, w2h_ref, w2l_ref, b2_ref, s_ref, o_ref,
             eu_ref, en_ref):
        e_in = e_ref[...]
        xc = jnp.concatenate([gs_ref[...], gr_ref[...], e_in], axis=-1)
        x = _d3(xc, w0h_ref[...], w0l_ref[...]) + b0_ref[...]
        eu = _mlp_tail(x, w1h_ref[...], w1l_ref[...], b1_ref[...],
                       w2h_ref[...], w2l_ref[...], b2_ref[...],
                       s_ref[...], o_ref[...])
        eu_ref[...] = eu
        en_ref[...] = e_in + eu

    args = (gs, gr, e, w0e['h'], w0e['l'], b0, w1['h'], w1['l'], b1, w2['h'],
            w2['l'], b2, ln_s, ln_o)
    grid = (N_EDGES // _BE,)
    return pl.pallas_call(
        body,
        grid=grid,
        in_specs=[_row_spec(_BE, LATENT)] * 3 + [
            _full_spec(a.shape) for a in args[3:]],
        out_specs=[_row_spec(_BE, LATENT)] * 2,
        out_shape=[jax.ShapeDtypeStruct((N_EDGES, LATENT), F32)] * 2,
    )(*args)


def _node_call(h, a0, a1, w0n, b0, w1, b1, w2, b2, ln_s, ln_o):
    def body(h_ref, a0_ref, a1_ref, w0h_ref, w0l_ref, b0_ref, w1h_ref,
             w1l_ref, b1_ref, w2h_ref, w2l_ref, b2_ref, s_ref, o_ref,
             hn_ref):
        hh = h_ref[...]
        agg = a0_ref[...] + a1_ref[...]
        xc = jnp.concatenate([hh, agg], axis=-1)
        x = _d3(xc, w0h_ref[...], w0l_ref[...]) + b0_ref[...]
        nu = _mlp_tail(x, w1h_ref[...], w1l_ref[...], b1_ref[...],
                       w2h_ref[...], w2l_ref[...], b2_ref[...],
                       s_ref[...], o_ref[...])
        hn_ref[...] = hh + nu

    args = (h, a0, a1, w0n['h'], w0n['l'], b0, w1['h'], w1['l'], b1,
            w2['h'], w2['l'], b2, ln_s, ln_o)
    grid = (N_NODES // _BN,)
    return pl.pallas_call(
        body,
        grid=grid,
        in_specs=[_row_spec(_BN, LATENT)] * 3 + [
            _full_spec(a.shape) for a in args[3:]],
        out_specs=_row_spec(_BN, LATENT),
        out_shape=jax.ShapeDtypeStruct((N_NODES, LATENT), F32),
    )(*args)


def _dec_call(h, w0, b0, w1, b1, w2, b2):
    def body(h_ref, w0h_ref, w0l_ref, b0_ref, w1h_ref, w1l_ref, b1_ref,
             w2h_ref, w2l_ref, b2_ref, out_ref):
        x = _d3(h_ref[...], w0h_ref[...], w0l_ref[...]) + b0_ref[...]
        x = jnp.maximum(x, 0.0)
        x = _d3(x, w1h_ref[...], w1l_ref[...]) + b1_ref[...]
        x = jnp.maximum(x, 0.0)
        out_ref[...] = _d3(x, w2h_ref[...], w2l_ref[...]) + b2_ref[...]

    args = (h, w0['h'], w0['l'], b0, w1['h'], w1['l'], b1, w2['h'], w2['l'],
            b2)
    grid = (N_NODES // _BN,)
    return pl.pallas_call(
        body,
        grid=grid,
        in_specs=[_row_spec(_BN, LATENT)] + [
            _full_spec(a.shape) for a in args[1:]],
        out_specs=_row_spec(_BN, 3),
        out_shape=jax.ShapeDtypeStruct((N_NODES, 3), F32),
    )(*args)


def _sc_mesh():
    return plsc.VectorSubcoreMesh(core_axis_name="c", subcore_axis_name="s",
                                  num_cores=NC, num_subcores=NS)

_EPW = N_EDGES // NW
_GC = 80
_NB = 5
_GG = _EPW // (_GC * _NB)
_SNB = 4
_SGG = _EPW // (_GC * _SNB)

_EPC = N_EDGES // NC
_NG = N_NODES // 8
_GPT = -(-_NG // NS)


def _gather2_call(ps, pr, senders, receivers):
    @functools.partial(
        pl.kernel,
        out_type=(jax.ShapeDtypeStruct((N_EDGES, LATENT), F32),) * 2,
        mesh=_sc_mesh(),
        scratch_types=[
            pltpu.VMEM((_NB, _GC), jnp.int32),
            pltpu.VMEM((_NB, _GC), jnp.int32),
            pltpu.VMEM((_NB, _GC, LATENT), F32),
            pltpu.VMEM((_NB, _GC, LATENT), F32),
        ] + [pltpu.SemaphoreType.DMA] * (3 * _NB),
    )
    def k(ps_hbm, pr_hbm, s_hbm, r_hbm, gs_hbm, gr_hbm, si, ri, srow, rrow,
          *sems):
        sem_i, sem_g, sem_o = sems[:_NB], sems[_NB:2 * _NB], sems[2 * _NB:]
        wid = lax.axis_index("s") * NC + lax.axis_index("c")
        base = wid * _EPW

        def body(g, carry):
            off0 = base + g * (_GC * _NB)
            di = []
            for b in range(_NB):
                off = off0 + b * _GC
                di.append((
                    pltpu.async_copy(s_hbm.at[pl.ds(off, _GC)], si.at[b],
                                     sem_i[b]),
                    pltpu.async_copy(r_hbm.at[pl.ds(off, _GC)], ri.at[b],
                                     sem_i[b]),
                ))
            dg = []
            for b in range(_NB):
                di[b][0].wait()
                di[b][1].wait()
                dg.append((
                    pltpu.async_copy(ps_hbm.at[si.at[b]], srow.at[b], sem_g[b]),
                    pltpu.async_copy(pr_hbm.at[ri.at[b]], rrow.at[b], sem_g[b]),
                ))
            do = []
            for b in range(_NB):
                off = off0 + b * _GC
                dg[b][0].wait()
                dg[b][1].wait()
                do.append((
                    pltpu.async_copy(srow.at[b], gs_hbm.at[pl.ds(off, _GC)],
                                     sem_o[b]),
                    pltpu.async_copy(rrow.at[b], gr_hbm.at[pl.ds(off, _GC)],
                                     sem_o[b]),
                ))
            for b in range(_NB):
                do[b][0].wait()
                do[b][1].wait()
            return carry

        lax.fori_loop(0, _GG, body, 0)

    return k(ps, pr, senders, receivers)


def _scatter_call(e_upd, receivers, zrows):
    @functools.partial(
        pl.kernel,
        out_type=jax.ShapeDtypeStruct((NC, N_NODES, LATENT), F32),
        mesh=_sc_mesh(),
        scratch_types=[
            pltpu.VMEM((_SNB, _GC), jnp.int32),
            pltpu.VMEM((_SNB, _GC, LATENT), F32),
            pltpu.VMEM_SHARED((N_NODES, LATENT), F32),
        ] + [pltpu.SemaphoreType.DMA] * (2 * _SNB),
    )
    def k(e_hbm, r_hbm, z_hbm, out_hbm, idx, row, shared, *sems):
        sem_i, sem_a = sems[:_SNB], sems[_SNB:]
        cid = lax.axis_index("c")
        sid = lax.axis_index("s")

        def zbody(kk, carry):
            g = sid * _GPT + kk

            @pl.when(g < _NG)
            def _():
                pltpu.sync_copy(z_hbm, shared.at[pl.ds(g * 8, 8)])

            return carry

        lax.fori_loop(0, _GPT, zbody, 0)
        plsc.subcore_barrier()
        base = cid * _EPC + sid * _EPW

        def body(g, carry):
            off0 = base + g * (_GC * _SNB)
            di = []
            for b in range(_SNB):
                off = off0 + b * _GC
                di.append((
                    pltpu.async_copy(r_hbm.at[pl.ds(off, _GC)], idx.at[b],
                                     sem_i[b]),
                    pltpu.async_copy(e_hbm.at[pl.ds(off, _GC)], row.at[b],
                                     sem_i[b]),
                ))
            da = []
            for b in range(_SNB):
                di[b][0].wait()
                di[b][1].wait()
                da.append(pltpu.async_copy(row.at[b], shared.at[idx.at[b]],
                                           sem_a[b], add=True))
            for b in range(_SNB):
                da[b].wait()
            return carry

        lax.fori_loop(0, _SGG, body, 0)
        for tc in range(_SGG * _SNB, _EPW // _GC):
            off = base + tc * _GC
            pltpu.sync_copy(r_hbm.at[pl.ds(off, _GC)], idx.at[0])
            pltpu.sync_copy(e_hbm.at[pl.ds(off, _GC)], row.at[0])
            pltpu.sync_copy(row.at[0], shared.at[idx.at[0]], add=True)
        plsc.subcore_barrier()

        def obody(kk, carry):
            g = sid * _GPT + kk

            @pl.when(g < _NG)
            def _():
                pltpu.sync_copy(shared.at[pl.ds(g * 8, 8)],
                                out_hbm.at[cid, pl.ds(g * 8, 8)])

            return carry

        lax.fori_loop(0, _GPT, obody, 0)

    return k(e_upd, receivers, zrows)


def kernel(nodes, edges, senders, receivers, particle_type, params):
    enc_n = params['enc_node']
    enc_e = params['enc_edge']
    dec = params['dec']

    def r2(b):
        return b.reshape(1, -1)

    pt2d = particle_type.reshape(N_NODES, 1)

    h = _enc_node_call(nodes, pt2d, _wsplit(enc_n['W0']), params['embed'],
                       r2(enc_n['b0']), _wsplit(enc_n['W1']), r2(enc_n['b1']),
                       _wsplit(enc_n['W2']), r2(enc_n['b2']),
                       r2(enc_n['ln_s']), r2(enc_n['ln_o']))
    e = _enc_edge_call(edges, _wsplit(enc_e['W0']), r2(enc_e['b0']),
                       _wsplit(enc_e['W1']), r2(enc_e['b1']),
                       _wsplit(enc_e['W2']), r2(enc_e['b2']),
                       r2(enc_e['ln_s']), r2(enc_e['ln_o']))

    zrows = jnp.zeros((8, LATENT), F32)

    for t in range(MP_STEPS):
        pe = params['proc_edge'][t]
        pn = params['proc_node'][t]
        gs, gr = _gather2_call(h, h, senders, receivers)
        e_upd, e = _edge_call(gs, gr, e, _wsplit(pe['W0']), r2(pe['b0']),
                              _wsplit(pe['W1']), r2(pe['b1']),
                              _wsplit(pe['W2']), r2(pe['b2']),
                              r2(pe['ln_s']), r2(pe['ln_o']))
        partials = _scatter_call(e_upd, receivers, zrows)
        h = _node_call(h, partials[0], partials[1], _wsplit(pn['W0']),
                       r2(pn['b0']), _wsplit(pn['W1']), r2(pn['b1']),
                       _wsplit(pn['W2']), r2(pn['b2']),
                       r2(pn['ln_s']), r2(pn['ln_o']))

    return _dec_call(h, _wsplit(dec['W0']), r2(dec['b0']), _wsplit(dec['W1']),
                     r2(dec['b1']), _wsplit(dec['W2']), r2(dec['b2']))


# speedup vs baseline: 1.1076x; 1.1076x over previous
import functools

import jax
import jax.numpy as jnp
from jax import lax
from jax.experimental import pallas as pl
from jax.experimental.pallas import tpu as pltpu
from jax.experimental.pallas import tpu_sc as plsc

N_NODES = 10000
N_EDGES = 320000
LATENT = 128
MP_STEPS = 10
N_TYPES = 9

NC, NS = 2, 16
NW = NC * NS

F32 = jnp.float32


BF16 = jnp.bfloat16


def _split(x):
    hi = x.astype(BF16)
    lo = (x - hi.astype(F32)).astype(BF16)
    return hi, lo


def _wsplit(w):
    hi, lo = _split(w)
    return {'h': hi, 'l': lo}


def _d3(x, wh, wl):
    xh, xl = _split(x)
    d = lambda a, b: jnp.dot(a, b, preferred_element_type=F32)
    return d(xh, wh) + (d(xh, wl) + d(xl, wh))


def _mlp_tail(x, w1h, w1l, b1, w2h, w2l, b2, ln_s, ln_o):
    x = jnp.maximum(x, 0.0)
    x = _d3(x, w1h, w1l) + b1
    x = jnp.maximum(x, 0.0)
    x = _d3(x, w2h, w2l) + b2
    mu = jnp.mean(x, axis=-1, keepdims=True)
    d = x - mu
    var = jnp.mean(d * d, axis=-1, keepdims=True)
    return d * lax.rsqrt(var + 1e-5) * ln_s + ln_o


def _row_spec(block, cols):
    return pl.BlockSpec((block, cols), lambda i: (i, 0))


def _full_spec(shape):
    nd = len(shape)
    return pl.BlockSpec(shape, lambda i: (0,) * nd)


_BN = 2000
_BE = 2000


def _enc_node_call(nodes, pt2d, w0n, te, b0, w1, b1, w2, b2, ln_s, ln_o):
    def body(n_ref, pt_ref, w0h_ref, w0l_ref, te_ref, b0_ref, w1h_ref, w1l_ref,
             b1_ref, w2h_ref, w2l_ref, b2_ref, s_ref, o_ref, h_ref):
        x = _d3(n_ref[...], w0h_ref[...], w0l_ref[...]) + b0_ref[...]
        pt = pt_ref[...]
        te = te_ref[...]
        emb = jnp.zeros_like(x)
        for t in range(N_TYPES):
            emb = jnp.where(pt == t, te[t][None, :], emb)
        x = x + emb
        h_ref[...] = _mlp_tail(x, w1h_ref[...], w1l_ref[...], b1_ref[...],
                               w2h_ref[...], w2l_ref[...], b2_ref[...],
                               s_ref[...], o_ref[...])

    args = (nodes, pt2d, w0n['h'], w0n['l'], te, b0, w1['h'], w1['l'], b1,
            w2['h'], w2['l'], b2, ln_s, ln_o)
    grid = (N_NODES // _BN,)
    return pl.pallas_call(
        body,
        grid=grid,
        in_specs=[_row_spec(_BN, 128), _row_spec(_BN, 1)] + [
            _full_spec(a.shape) for a in args[2:]],
        out_specs=_row_spec(_BN, LATENT),
        out_shape=jax.ShapeDtypeStruct((N_NODES, LATENT), F32),
    )(*args)


def _enc_edge_call(edges, w0, b0, w1, b1, w2, b2, ln_s, ln_o):
    def body(e_ref, w0h_ref, w0l_ref, b0_ref, w1h_ref, w1l_ref, b1_ref,
             w2h_ref, w2l_ref, b2_ref, s_ref, o_ref, out_ref):
        x = _d3(e_ref[...], w0h_ref[...], w0l_ref[...]) + b0_ref[...]
        out_ref[...] = _mlp_tail(x, w1h_ref[...], w1l_ref[...], b1_ref[...],
                                 w2h_ref[...], w2l_ref[...], b2_ref[...],
                                 s_ref[...], o_ref[...])

    args = (edges, w0['h'], w0['l'], b0, w1['h'], w1['l'], b1, w2['h'],
            w2['l'], b2, ln_s, ln_o)
    grid = (N_EDGES // _BE,)
    return pl.pallas_call(
        body,
        grid=grid,
        in_specs=[_row_spec(_BE, 16)] + [_full_spec(a.shape) for a in args[1:]],
        out_specs=_row_spec(_BE, LATENT),
        out_shape=jax.ShapeDtypeStruct((N_EDGES, LATENT), F32),
    )(*args)


def _proj_call(h, w0s, w0r, b0):
    def body(h_ref, wsh_ref, wsl_ref, wrh_ref, wrl_ref, b0_ref, ps_ref, pr_ref):
        hh = h_ref[...]
        ps_ref[...] = _d3(hh, wsh_ref[...], wsl_ref[...]) + b0_ref[...]
        pr_ref[...] = _d3(hh, wrh_ref[...], wrl_ref[...])

    args = (h, w0s['h'], w0s['l'], w0r['h'], w0r['l'], b0)
    grid = (N_NODES // _BN,)
    return pl.pallas_call(
        body,
        grid=grid,
        in_specs=[_row_spec(_BN, LATENT)] + [
            _full_spec(a.shape) for a in args[1:]],
        out_specs=[_row_spec(_BN, LATENT), _row_spec(_BN, LATENT)],
        out_shape=[jax.ShapeDtypeStruct((N_NODES, LATENT), F32)] * 2,
    )(*args)


def _edge_call(gs, gr, e, w0e, w1, b1, w2, b2, ln_s, ln_o):
    def body(gs_ref, gr_ref, e_ref, w0h_ref, w0l_ref, w1h_ref, w1l_ref,
             b1_ref, w2h_ref, w2l_ref, b2_ref, s_ref, o_ref, eu_ref, en_ref):
        e_in = e_ref[...]
        x = gs_ref[...] + gr_ref[...] + _d3(e_in, w0h_ref[...], w0l_ref[...])
        eu = _mlp_tail(x, w1h_ref[...], w1l_ref[...], b1_ref[...],
                       w2h_ref[...], w2l_ref[...], b2_ref[...],
                       s_ref[...], o_ref[...])
        eu_ref[...] = eu
        en_ref[...] = e_in + eu

    args = (gs, gr, e, w0e['h'], w0e['l'], w1['h'], w1['l'], b1, w2['h'],
            w2['l'], b2, ln_s, ln_o)
    grid = (N_EDGES // _BE,)
    return pl.pallas_call(
        body,
        grid=grid,
        in_specs=[_row_spec(_BE, LATENT)] * 3 + [
            _full_spec(a.shape) for a in args[3:]],
        out_specs=[_row_spec(_BE, LATENT)] * 2,
        out_shape=[jax.ShapeDtypeStruct((N_EDGES, LATENT), F32)] * 2,
    )(*args)


def _node_call(h, a0, a1, w0h, w0a, b0, w1, b1, w2, b2, ln_s, ln_o):
    def body(h_ref, a0_ref, a1_ref, whh_ref, whl_ref, wah_ref, wal_ref,
             b0_ref, w1h_ref, w1l_ref, b1_ref, w2h_ref, w2l_ref, b2_ref,
             s_ref, o_ref, hn_ref):
        hh = h_ref[...]
        agg = a0_ref[...] + a1_ref[...]
        x = (_d3(hh, whh_ref[...], whl_ref[...]) +
             _d3(agg, wah_ref[...], wal_ref[...]) + b0_ref[...])
        nu = _mlp_tail(x, w1h_ref[...], w1l_ref[...], b1_ref[...],
                       w2h_ref[...], w2l_ref[...], b2_ref[...],
                       s_ref[...], o_ref[...])
        hn_ref[...] = hh + nu

    args = (h, a0, a1, w0h['h'], w0h['l'], w0a['h'], w0a['l'], b0, w1['h'],
            w1['l'], b1, w2['h'], w2['l'], b2, ln_s, ln_o)
    grid = (N_NODES // _BN,)
    return pl.pallas_call(
        body,
        grid=grid,
        in_specs=[_row_spec(_BN, LATENT)] * 3 + [
            _full_spec(a.shape) for a in args[3:]],
        out_specs=_row_spec(_BN, LATENT),
        out_shape=jax.ShapeDtypeStruct((N_NODES, LATENT), F32),
    )(*args)


def _dec_call(h, w0, b0, w1, b1, w2, b2):
    def body(h_ref, w0h_ref, w0l_ref, b0_ref, w1h_ref, w1l_ref, b1_ref,
             w2h_ref, w2l_ref, b2_ref, out_ref):
        x = _d3(h_ref[...], w0h_ref[...], w0l_ref[...]) + b0_ref[...]
        x = jnp.maximum(x, 0.0)
        x = _d3(x, w1h_ref[...], w1l_ref[...]) + b1_ref[...]
        x = jnp.maximum(x, 0.0)
        out_ref[...] = _d3(x, w2h_ref[...], w2l_ref[...]) + b2_ref[...]

    args = (h, w0['h'], w0['l'], b0, w1['h'], w1['l'], b1, w2['h'], w2['l'],
            b2)
    grid = (N_NODES // _BN,)
    return pl.pallas_call(
        body,
        grid=grid,
        in_specs=[_row_spec(_BN, LATENT)] + [
            _full_spec(a.shape) for a in args[1:]],
        out_specs=_row_spec(_BN, 3),
        out_shape=jax.ShapeDtypeStruct((N_NODES, 3), F32),
    )(*args)


def _sc_mesh():
    return plsc.VectorSubcoreMesh(core_axis_name="c", subcore_axis_name="s",
                                  num_cores=NC, num_subcores=NS)

_EPW = N_EDGES // NW
_GC = 80
_NB = 5
_GG = _EPW // (_GC * _NB)
_SNB = 4
_SGG = _EPW // (_GC * _SNB)

_EPC = N_EDGES // NC
_NG = N_NODES // 8
_GPT = -(-_NG // NS)


def _gather2_call(ps, pr, senders, receivers):
    @functools.partial(
        pl.kernel,
        out_type=(jax.ShapeDtypeStruct((N_EDGES, LATENT), F32),) * 2,
        mesh=_sc_mesh(),
        scratch_types=[
            pltpu.VMEM((_NB, _GC), jnp.int32),
            pltpu.VMEM((_NB, _GC), jnp.int32),
            pltpu.VMEM((_NB, _GC, LATENT), F32),
            pltpu.VMEM((_NB, _GC, LATENT), F32),
        ] + [pltpu.SemaphoreType.DMA] * (3 * _NB),
    )
    def k(ps_hbm, pr_hbm, s_hbm, r_hbm, gs_hbm, gr_hbm, si, ri, srow, rrow,
          *sems):
        sem_i, sem_g, sem_o = sems[:_NB], sems[_NB:2 * _NB], sems[2 * _NB:]
        wid = lax.axis_index("s") * NC + lax.axis_index("c")
        base = wid * _EPW

        def body(g, carry):
            off0 = base + g * (_GC * _NB)
            di = []
            for b in range(_NB):
                off = off0 + b * _GC
                di.append((
                    pltpu.async_copy(s_hbm.at[pl.ds(off, _GC)], si.at[b],
                                     sem_i[b]),
                    pltpu.async_copy(r_hbm.at[pl.ds(off, _GC)], ri.at[b],
                                     sem_i[b]),
                ))
            dg = []
            for b in range(_NB):
                di[b][0].wait()
                di[b][1].wait()
                dg.append((
                    pltpu.async_copy(ps_hbm.at[si.at[b]], srow.at[b], sem_g[b]),
                    pltpu.async_copy(pr_hbm.at[ri.at[b]], rrow.at[b], sem_g[b]),
                ))
            do = []
            for b in range(_NB):
                off = off0 + b * _GC
                dg[b][0].wait()
                dg[b][1].wait()
                do.append((
                    pltpu.async_copy(srow.at[b], gs_hbm.at[pl.ds(off, _GC)],
                                     sem_o[b]),
                    pltpu.async_copy(rrow.at[b], gr_hbm.at[pl.ds(off, _GC)],
                                     sem_o[b]),
                ))
            for b in range(_NB):
                do[b][0].wait()
                do[b][1].wait()
            return carry

        lax.fori_loop(0, _GG, body, 0)

    return k(ps, pr, senders, receivers)


def _scatter_call(e_upd, receivers, zrows):
    @functools.partial(
        pl.kernel,
        out_type=jax.ShapeDtypeStruct((NC, N_NODES, LATENT), F32),
        mesh=_sc_mesh(),
        scratch_types=[
            pltpu.VMEM((_SNB, _GC), jnp.int32),
            pltpu.VMEM((_SNB, _GC, LATENT), F32),
            pltpu.VMEM_SHARED((N_NODES, LATENT), F32),
        ] + [pltpu.SemaphoreType.DMA] * (2 * _SNB),
    )
    def k(e_hbm, r_hbm, z_hbm, out_hbm, idx, row, shared, *sems):
        sem_i, sem_a = sems[:_SNB], sems[_SNB:]
        cid = lax.axis_index("c")
        sid = lax.axis_index("s")

        def zbody(kk, carry):
            g = sid * _GPT + kk

            @pl.when(g < _NG)
            def _():
                pltpu.sync_copy(z_hbm, shared.at[pl.ds(g * 8, 8)])

            return carry

        lax.fori_loop(0, _GPT, zbody, 0)
        plsc.subcore_barrier()
        base = cid * _EPC + sid * _EPW

        def body(g, carry):
            off0 = base + g * (_GC * _SNB)
            di = []
            for b in range(_SNB):
                off = off0 + b * _GC
                di.append((
                    pltpu.async_copy(r_hbm.at[pl.ds(off, _GC)], idx.at[b],
                                     sem_i[b]),
                    pltpu.async_copy(e_hbm.at[pl.ds(off, _GC)], row.at[b],
                                     sem_i[b]),
                ))
            da = []
            for b in range(_SNB):
                di[b][0].wait()
                di[b][1].wait()
                da.append(pltpu.async_copy(row.at[b], shared.at[idx.at[b]],
                                           sem_a[b], add=True))
            for b in range(_SNB):
                da[b].wait()
            return carry

        lax.fori_loop(0, _SGG, body, 0)
        for tc in range(_SGG * _SNB, _EPW // _GC):
            off = base + tc * _GC
            pltpu.sync_copy(r_hbm.at[pl.ds(off, _GC)], idx.at[0])
            pltpu.sync_copy(e_hbm.at[pl.ds(off, _GC)], row.at[0])
            pltpu.sync_copy(row.at[0], shared.at[idx.at[0]], add=True)
        plsc.subcore_barrier()

        def obody(kk, carry):
            g = sid * _GPT + kk

            @pl.when(g < _NG)
            def _():
                pltpu.sync_copy(shared.at[pl.ds(g * 8, 8)],
                                out_hbm.at[cid, pl.ds(g * 8, 8)])

            return carry

        lax.fori_loop(0, _GPT, obody, 0)

    return k(e_upd, receivers, zrows)


def kernel(nodes, edges, senders, receivers, particle_type, params):
    enc_n = params['enc_node']
    enc_e = params['enc_edge']
    dec = params['dec']

    def r2(b):
        return b.reshape(1, -1)

    te = jnp.dot(params['embed'], enc_n['W0'][128:])
    pt2d = particle_type.reshape(N_NODES, 1)

    h = _enc_node_call(nodes, pt2d, _wsplit(enc_n['W0'][:128]), te,
                       r2(enc_n['b0']), _wsplit(enc_n['W1']), r2(enc_n['b1']),
                       _wsplit(enc_n['W2']), r2(enc_n['b2']),
                       r2(enc_n['ln_s']), r2(enc_n['ln_o']))
    e = _enc_edge_call(edges, _wsplit(enc_e['W0']), r2(enc_e['b0']),
                       _wsplit(enc_e['W1']), r2(enc_e['b1']),
                       _wsplit(enc_e['W2']), r2(enc_e['b2']),
                       r2(enc_e['ln_s']), r2(enc_e['ln_o']))

    zrows = jnp.zeros((8, LATENT), F32)

    for t in range(MP_STEPS):
        pe = params['proc_edge'][t]
        pn = params['proc_node'][t]
        w0 = pe['W0']
        ps, pr = _proj_call(h, _wsplit(w0[:LATENT]),
                            _wsplit(w0[LATENT:2 * LATENT]), r2(pe['b0']))
        gs, gr = _gather2_call(ps, pr, senders, receivers)
        e_upd, e = _edge_call(gs, gr, e, _wsplit(w0[2 * LATENT:]),
                              _wsplit(pe['W1']), r2(pe['b1']),
                              _wsplit(pe['W2']), r2(pe['b2']),
                              r2(pe['ln_s']), r2(pe['ln_o']))
        partials = _scatter_call(e_upd, receivers, zrows)
        wn0 = pn['W0']
        h = _node_call(h, partials[0], partials[1], _wsplit(wn0[:LATENT]),
                       _wsplit(wn0[LATENT:]), r2(pn['b0']), _wsplit(pn['W1']),
                       r2(pn['b1']), _wsplit(pn['W2']), r2(pn['b2']),
                       r2(pn['ln_s']), r2(pn['ln_o']))

    return _dec_call(h, _wsplit(dec['W0']), r2(dec['b0']), _wsplit(dec['W1']),
                     r2(dec['b1']), _wsplit(dec['W2']), r2(dec['b2']))


# speedup vs baseline: 1.1189x; 1.0102x over previous
import functools

import jax
import jax.numpy as jnp
from jax import lax
from jax.experimental import pallas as pl
from jax.experimental.pallas import tpu as pltpu
from jax.experimental.pallas import tpu_sc as plsc

N_NODES = 10000
N_EDGES = 320000
LATENT = 128
MP_STEPS = 10
N_TYPES = 9

NC, NS = 2, 16
NW = NC * NS

F32 = jnp.float32


BF16 = jnp.bfloat16


def _split(x):
    hi = x.astype(BF16)
    lo = (x - hi.astype(F32)).astype(BF16)
    return hi, lo


def _wsplit(w):
    hi, lo = _split(w)
    return {'h': hi, 'l': lo}


def _d3(x, wh, wl):
    xh, xl = _split(x)
    d = lambda a, b: jnp.dot(a, b, preferred_element_type=F32)
    return d(xh, wh) + (d(xh, wl) + d(xl, wh))


def _mlp_tail(x, w1h, w1l, b1, w2h, w2l, b2, ln_s, ln_o):
    x = jnp.maximum(x, 0.0)
    x = _d3(x, w1h, w1l) + b1
    x = jnp.maximum(x, 0.0)
    x = _d3(x, w2h, w2l) + b2
    mu = jnp.mean(x, axis=-1, keepdims=True)
    d = x - mu
    var = jnp.mean(d * d, axis=-1, keepdims=True)
    return d * lax.rsqrt(var + 1e-5) * ln_s + ln_o


def _row_spec(block, cols):
    return pl.BlockSpec((block, cols), lambda i: (i, 0))


def _full_spec(shape):
    nd = len(shape)
    return pl.BlockSpec(shape, lambda i: (0,) * nd)


_BN = 2000
_BE = 2000


def _enc_node_call(nodes, pt2d, w0n, te, b0, w1, b1, w2, b2, ln_s, ln_o):
    def body(n_ref, pt_ref, w0h_ref, w0l_ref, te_ref, b0_ref, w1h_ref, w1l_ref,
             b1_ref, w2h_ref, w2l_ref, b2_ref, s_ref, o_ref, h_ref):
        x = _d3(n_ref[...], w0h_ref[...], w0l_ref[...]) + b0_ref[...]
        pt = pt_ref[...]
        te = te_ref[...]
        emb = jnp.zeros_like(x)
        for t in range(N_TYPES):
            emb = jnp.where(pt == t, te[t][None, :], emb)
        x = x + emb
        h_ref[...] = _mlp_tail(x, w1h_ref[...], w1l_ref[...], b1_ref[...],
                               w2h_ref[...], w2l_ref[...], b2_ref[...],
                               s_ref[...], o_ref[...])

    args = (nodes, pt2d, w0n['h'], w0n['l'], te, b0, w1['h'], w1['l'], b1,
            w2['h'], w2['l'], b2, ln_s, ln_o)
    grid = (N_NODES // _BN,)
    return pl.pallas_call(
        body,
        grid=grid,
        in_specs=[_row_spec(_BN, 128), _row_spec(_BN, 1)] + [
            _full_spec(a.shape) for a in args[2:]],
        out_specs=_row_spec(_BN, LATENT),
        out_shape=jax.ShapeDtypeStruct((N_NODES, LATENT), F32),
    )(*args)


def _enc_edge_call(edges, w0, b0, w1, b1, w2, b2, ln_s, ln_o):
    def body(e_ref, w0h_ref, w0l_ref, b0_ref, w1h_ref, w1l_ref, b1_ref,
             w2h_ref, w2l_ref, b2_ref, s_ref, o_ref, out_ref):
        x = _d3(e_ref[...], w0h_ref[...], w0l_ref[...]) + b0_ref[...]
        out_ref[...] = _mlp_tail(x, w1h_ref[...], w1l_ref[...], b1_ref[...],
                                 w2h_ref[...], w2l_ref[...], b2_ref[...],
                                 s_ref[...], o_ref[...])

    args = (edges, w0['h'], w0['l'], b0, w1['h'], w1['l'], b1, w2['h'],
            w2['l'], b2, ln_s, ln_o)
    grid = (N_EDGES // _BE,)
    return pl.pallas_call(
        body,
        grid=grid,
        in_specs=[_row_spec(_BE, 16)] + [_full_spec(a.shape) for a in args[1:]],
        out_specs=_row_spec(_BE, LATENT),
        out_shape=jax.ShapeDtypeStruct((N_EDGES, LATENT), F32),
    )(*args)


def _proj_call(h, w0s, w0r, b0):
    def body(h_ref, wsh_ref, wsl_ref, wrh_ref, wrl_ref, b0_ref, ps_ref, pr_ref):
        hh = h_ref[...]
        ps_ref[...] = _d3(hh, wsh_ref[...], wsl_ref[...]) + b0_ref[...]
        pr_ref[...] = _d3(hh, wrh_ref[...], wrl_ref[...])

    args = (h, w0s['h'], w0s['l'], w0r['h'], w0r['l'], b0)
    grid = (N_NODES // _BN,)
    return pl.pallas_call(
        body,
        grid=grid,
        in_specs=[_row_spec(_BN, LATENT)] + [
            _full_spec(a.shape) for a in args[1:]],
        out_specs=[_row_spec(_BN, LATENT), _row_spec(_BN, LATENT)],
        out_shape=[jax.ShapeDtypeStruct((N_NODES, LATENT), F32)] * 2,
    )(*args)


def _edge_call(g, e, w0e, w1, b1, w2, b2, ln_s, ln_o):
    def body(g_ref, e_ref, w0h_ref, w0l_ref, w1h_ref, w1l_ref,
             b1_ref, w2h_ref, w2l_ref, b2_ref, s_ref, o_ref, eu_ref, en_ref):
        e_in = e_ref[...]
        x = g_ref[...] + _d3(e_in, w0h_ref[...], w0l_ref[...])
        eu = _mlp_tail(x, w1h_ref[...], w1l_ref[...], b1_ref[...],
                       w2h_ref[...], w2l_ref[...], b2_ref[...],
                       s_ref[...], o_ref[...])
        eu_ref[...] = eu
        en_ref[...] = e_in + eu

    args = (g, e, w0e['h'], w0e['l'], w1['h'], w1['l'], b1, w2['h'],
            w2['l'], b2, ln_s, ln_o)
    grid = (N_EDGES // _BE,)
    return pl.pallas_call(
        body,
        grid=grid,
        in_specs=[_row_spec(_BE, LATENT)] * 2 + [
            _full_spec(a.shape) for a in args[2:]],
        out_specs=[_row_spec(_BE, LATENT)] * 2,
        out_shape=[jax.ShapeDtypeStruct((N_EDGES, LATENT), F32)] * 2,
    )(*args)


def _node_call(h, a0, a1, w0h, w0a, b0, w1, b1, w2, b2, ln_s, ln_o):
    def body(h_ref, a0_ref, a1_ref, whh_ref, whl_ref, wah_ref, wal_ref,
             b0_ref, w1h_ref, w1l_ref, b1_ref, w2h_ref, w2l_ref, b2_ref,
             s_ref, o_ref, hn_ref):
        hh = h_ref[...]
        agg = a0_ref[...] + a1_ref[...]
        x = (_d3(hh, whh_ref[...], whl_ref[...]) +
             _d3(agg, wah_ref[...], wal_ref[...]) + b0_ref[...])
        nu = _mlp_tail(x, w1h_ref[...], w1l_ref[...], b1_ref[...],
                       w2h_ref[...], w2l_ref[...], b2_ref[...],
                       s_ref[...], o_ref[...])
        hn_ref[...] = hh + nu

    args = (h, a0, a1, w0h['h'], w0h['l'], w0a['h'], w0a['l'], b0, w1['h'],
            w1['l'], b1, w2['h'], w2['l'], b2, ln_s, ln_o)
    grid = (N_NODES // _BN,)
    return pl.pallas_call(
        body,
        grid=grid,
        in_specs=[_row_spec(_BN, LATENT)] * 3 + [
            _full_spec(a.shape) for a in args[3:]],
        out_specs=_row_spec(_BN, LATENT),
        out_shape=jax.ShapeDtypeStruct((N_NODES, LATENT), F32),
    )(*args)


def _dec_call(h, w0, b0, w1, b1, w2, b2):
    def body(h_ref, w0h_ref, w0l_ref, b0_ref, w1h_ref, w1l_ref, b1_ref,
             w2h_ref, w2l_ref, b2_ref, out_ref):
        x = _d3(h_ref[...], w0h_ref[...], w0l_ref[...]) + b0_ref[...]
        x = jnp.maximum(x, 0.0)
        x = _d3(x, w1h_ref[...], w1l_ref[...]) + b1_ref[...]
        x = jnp.maximum(x, 0.0)
        out_ref[...] = _d3(x, w2h_ref[...], w2l_ref[...]) + b2_ref[...]

    args = (h, w0['h'], w0['l'], b0, w1['h'], w1['l'], b1, w2['h'], w2['l'],
            b2)
    grid = (N_NODES // _BN,)
    return pl.pallas_call(
        body,
        grid=grid,
        in_specs=[_row_spec(_BN, LATENT)] + [
            _full_spec(a.shape) for a in args[1:]],
        out_specs=_row_spec(_BN, 3),
        out_shape=jax.ShapeDtypeStruct((N_NODES, 3), F32),
    )(*args)


def _sc_mesh():
    return plsc.VectorSubcoreMesh(core_axis_name="c", subcore_axis_name="s",
                                  num_cores=NC, num_subcores=NS)

_EPW = N_EDGES // NW
_GC = 80
_NB = 5
_GG = _EPW // (_GC * _NB)
_SNB = 4
_SGG = _EPW // (_GC * _SNB)

_EPC = N_EDGES // NC
_NG = N_NODES // 8
_GPT = -(-_NG // NS)


def _gather2_call(ps, pr, senders, receivers):
    @functools.partial(
        pl.kernel,
        out_type=jax.ShapeDtypeStruct((N_EDGES, LATENT), F32),
        mesh=_sc_mesh(),
        scratch_types=[
            pltpu.VMEM((_NB, _GC), jnp.int32),
            pltpu.VMEM((_NB, _GC), jnp.int32),
            pltpu.VMEM((_NB, _GC, LATENT), F32),
            pltpu.VMEM((_NB, _GC, LATENT), F32),
        ] + [pltpu.SemaphoreType.DMA] * (3 * _NB),
    )
    def k(ps_hbm, pr_hbm, s_hbm, r_hbm, g_hbm, si, ri, srow, rrow, *sems):
        sem_i, sem_g, sem_o = sems[:_NB], sems[_NB:2 * _NB], sems[2 * _NB:]
        wid = lax.axis_index("s") * NC + lax.axis_index("c")
        base = wid * _EPW

        def body(g, carry):
            off0 = base + g * (_GC * _NB)
            di = []
            for b in range(_NB):
                off = off0 + b * _GC
                di.append((
                    pltpu.async_copy(s_hbm.at[pl.ds(off, _GC)], si.at[b],
                                     sem_i[b]),
                    pltpu.async_copy(r_hbm.at[pl.ds(off, _GC)], ri.at[b],
                                     sem_i[b]),
                ))
            dg = []
            for b in range(_NB):
                di[b][0].wait()
                di[b][1].wait()
                dg.append((
                    pltpu.async_copy(ps_hbm.at[si.at[b]], srow.at[b], sem_g[b]),
                    pltpu.async_copy(pr_hbm.at[ri.at[b]], rrow.at[b], sem_g[b]),
                ))
            do = []
            for b in range(_NB):
                off = off0 + b * _GC
                dg[b][0].wait()
                dg[b][1].wait()

                def add_row(i, c, _b=b):
                    for j in range(LATENT // 16):
                        sl = pl.ds(j * 16, 16)
                        srow[_b, i, sl] = srow[_b, i, sl] + rrow[_b, i, sl]
                    return c

                lax.fori_loop(0, _GC, add_row, 0)
                do.append(pltpu.async_copy(srow.at[b],
                                           g_hbm.at[pl.ds(off, _GC)],
                                           sem_o[b]))
            for b in range(_NB):
                do[b].wait()
            return carry

        lax.fori_loop(0, _GG, body, 0)

    return k(ps, pr, senders, receivers)


def _scatter_call(e_upd, receivers, zrows):
    @functools.partial(
        pl.kernel,
        out_type=jax.ShapeDtypeStruct((NC, N_NODES, LATENT), F32),
        mesh=_sc_mesh(),
        scratch_types=[
            pltpu.VMEM((_SNB, _GC), jnp.int32),
            pltpu.VMEM((_SNB, _GC, LATENT), F32),
            pltpu.VMEM_SHARED((N_NODES, LATENT), F32),
        ] + [pltpu.SemaphoreType.DMA] * (2 * _SNB),
    )
    def k(e_hbm, r_hbm, z_hbm, out_hbm, idx, row, shared, *sems):
        sem_i, sem_a = sems[:_SNB], sems[_SNB:]
        cid = lax.axis_index("c")
        sid = lax.axis_index("s")

        def zbody(kk, carry):
            g = sid * _GPT + kk

            @pl.when(g < _NG)
            def _():
                pltpu.sync_copy(z_hbm, shared.at[pl.ds(g * 8, 8)])

            return carry

        lax.fori_loop(0, _GPT, zbody, 0)
        plsc.subcore_barrier()
        base = cid * _EPC + sid * _EPW

        def body(g, carry):
            off0 = base + g * (_GC * _SNB)
            di = []
            for b in range(_SNB):
                off = off0 + b * _GC
                di.append((
                    pltpu.async_copy(r_hbm.at[pl.ds(off, _GC)], idx.at[b],
                                     sem_i[b]),
                    pltpu.async_copy(e_hbm.at[pl.ds(off, _GC)], row.at[b],
                                     sem_i[b]),
                ))
            da = []
            for b in range(_SNB):
                di[b][0].wait()
                di[b][1].wait()
                da.append(pltpu.async_copy(row.at[b], shared.at[idx.at[b]],
                                           sem_a[b], add=True))
            for b in range(_SNB):
                da[b].wait()
            return carry

        lax.fori_loop(0, _SGG, body, 0)
        for tc in range(_SGG * _SNB, _EPW // _GC):
            off = base + tc * _GC
            pltpu.sync_copy(r_hbm.at[pl.ds(off, _GC)], idx.at[0])
            pltpu.sync_copy(e_hbm.at[pl.ds(off, _GC)], row.at[0])
            pltpu.sync_copy(row.at[0], shared.at[idx.at[0]], add=True)
        plsc.subcore_barrier()

        def obody(kk, carry):
            g = sid * _GPT + kk

            @pl.when(g < _NG)
            def _():
                pltpu.sync_copy(shared.at[pl.ds(g * 8, 8)],
                                out_hbm.at[cid, pl.ds(g * 8, 8)])

            return carry

        lax.fori_loop(0, _GPT, obody, 0)

    return k(e_upd, receivers, zrows)


def kernel(nodes, edges, senders, receivers, particle_type, params):
    enc_n = params['enc_node']
    enc_e = params['enc_edge']
    dec = params['dec']

    def r2(b):
        return b.reshape(1, -1)

    te = jnp.dot(params['embed'], enc_n['W0'][128:])
    pt2d = particle_type.reshape(N_NODES, 1)

    h = _enc_node_call(nodes, pt2d, _wsplit(enc_n['W0'][:128]), te,
                       r2(enc_n['b0']), _wsplit(enc_n['W1']), r2(enc_n['b1']),
                       _wsplit(enc_n['W2']), r2(enc_n['b2']),
                       r2(enc_n['ln_s']), r2(enc_n['ln_o']))
    e = _enc_edge_call(edges, _wsplit(enc_e['W0']), r2(enc_e['b0']),
                       _wsplit(enc_e['W1']), r2(enc_e['b1']),
                       _wsplit(enc_e['W2']), r2(enc_e['b2']),
                       r2(enc_e['ln_s']), r2(enc_e['ln_o']))

    zrows = jnp.zeros((8, LATENT), F32)

    for t in range(MP_STEPS):
        pe = params['proc_edge'][t]
        pn = params['proc_node'][t]
        w0 = pe['W0']
        ps, pr = _proj_call(h, _wsplit(w0[:LATENT]),
                            _wsplit(w0[LATENT:2 * LATENT]), r2(pe['b0']))
        g = _gather2_call(ps, pr, senders, receivers)
        e_upd, e = _edge_call(g, e, _wsplit(w0[2 * LATENT:]),
                              _wsplit(pe['W1']), r2(pe['b1']),
                              _wsplit(pe['W2']), r2(pe['b2']),
                              r2(pe['ln_s']), r2(pe['ln_o']))
        partials = _scatter_call(e_upd, receivers, zrows)
        wn0 = pn['W0']
        h = _node_call(h, partials[0], partials[1], _wsplit(wn0[:LATENT]),
                       _wsplit(wn0[LATENT:]), r2(pn['b0']), _wsplit(pn['W1']),
                       r2(pn['b1']), _wsplit(pn['W2']), r2(pn['b2']),
                       r2(pn['ln_s']), r2(pn['ln_o']))

    return _dec_call(h, _wsplit(dec['W0']), r2(dec['b0']), _wsplit(dec['W1']),
                     r2(dec['b1']), _wsplit(dec['W2']), r2(dec['b2']))
